# baseline (speedup 1.0000x reference)
import jax
import jax.numpy as jnp
from jax.experimental import pallas as pl
from jax.experimental.pallas import tpu as pltpu

_T = 512


def _att_bag_kernel(bag_labels_ref, xa_ref, xb_ref, w_ref, b_ref,
                    repre_ref, logits_ref, wsc_ref, m_ref, s_ref, acc_ref):
    i = pl.program_id(0)
    j = pl.program_id(1)
    nj = pl.num_programs(1)
    C = w_ref.shape[0]
    H = w_ref.shape[1]

    @pl.when(j == 0)
    def _init():
        for k in range(2):
            lab = bag_labels_ref[2 * i + k]
            onehot = (jax.lax.broadcasted_iota(jnp.int32, (1, C), 1) == lab
                      ).astype(jnp.float32)
            wsc_ref[k:k + 1, :] = jax.lax.dot_general(
                onehot, w_ref[...], (((1,), (0,)), ((), ())),
                preferred_element_type=jnp.float32,
            )
        m_ref[...] = jnp.full(m_ref.shape, -jnp.inf, jnp.float32)
        s_ref[...] = jnp.zeros(s_ref.shape, jnp.float32)
        acc_ref[...] = jnp.zeros(acc_ref.shape, jnp.float32)

    for k, x_ref in enumerate((xa_ref, xb_ref)):
        x = x_ref[...]
        w = wsc_ref[k:k + 1, :]
        logit = jax.lax.dot_general(
            x, w, (((1,), (1,)), ((), ())), preferred_element_type=jnp.float32
        )
        m_old = m_ref[k:k + 1, 0:1]
        m_new = jnp.maximum(m_old, jnp.max(logit))
        corr = jnp.exp(m_old - m_new)
        p = jnp.exp(logit - m_new)
        m_ref[k:k + 1, 0:1] = m_new
        s_ref[k:k + 1, 0:1] = s_ref[k:k + 1, 0:1] * corr + jnp.sum(p)
        pool = jax.lax.dot_general(
            p, x, (((0,), (0,)), ((), ())), preferred_element_type=jnp.float32
        )
        acc_ref[k:k + 1, :] = acc_ref[k:k + 1, :] * corr + pool

    @pl.when(j == nj - 1)
    def _fini():
        reps, rows = [], []
        for k in range(2):
            repre = acc_ref[k:k + 1, :] * (1.0 / s_ref[k:k + 1, 0:1])
            row = jax.lax.dot_general(
                repre, w_ref[...], (((1,), (1,)), ((), ())),
                preferred_element_type=jnp.float32,
            ) + b_ref[...]
            reps.append(repre)
            rows.append(row)
        repre_ref[...] = jnp.concatenate(reps, axis=0).reshape(2, 1, H)
        logits_ref[...] = jnp.concatenate(rows, axis=0).reshape(2, 1, C)


def kernel(x, labels, scopes, W, b):
    N, H = x.shape
    C = W.shape[0]
    B = scopes.shape[0]
    L = N // B
    nj = L // _T

    starts = jnp.asarray(scopes)[:, 0].astype(jnp.int32)
    bag_labels = jnp.take(labels, starts, axis=0).astype(jnp.int32)
    b2 = b.reshape(1, C)

    grid_spec = pltpu.PrefetchScalarGridSpec(
        num_scalar_prefetch=1,
        grid=(B // 2, nj),
        in_specs=[
            pl.BlockSpec((_T, H), lambda i, j, *_: (2 * i * nj + j, 0)),
            pl.BlockSpec((_T, H), lambda i, j, *_: ((2 * i + 1) * nj + j, 0)),
            pl.BlockSpec((C, H), lambda i, j, *_: (0, 0)),
            pl.BlockSpec((1, C), lambda i, j, *_: (0, 0)),
        ],
        out_specs=[
            pl.BlockSpec((2, 1, H), lambda i, j, *_: (i, 0, 0)),
            pl.BlockSpec((2, 1, C), lambda i, j, *_: (i, 0, 0)),
        ],
        scratch_shapes=[
            pltpu.VMEM((2, H), jnp.float32),
            pltpu.VMEM((2, 128), jnp.float32),
            pltpu.VMEM((2, 128), jnp.float32),
            pltpu.VMEM((2, H), jnp.float32),
        ],
    )
    repre3, logits3 = pl.pallas_call(
        _att_bag_kernel,
        grid_spec=grid_spec,
        out_shape=[
            jax.ShapeDtypeStruct((B, 1, H), jnp.float32),
            jax.ShapeDtypeStruct((B, 1, C), jnp.float32),
        ],
        compiler_params=pltpu.CompilerParams(
            dimension_semantics=("parallel", "arbitrary")
        ),
    )(bag_labels, x, x, W, b2)
    return (repre3.reshape(B, H), logits3.reshape(B, C))


# speedup vs baseline: 1.1713x; 1.1713x over previous
import jax
import jax.numpy as jnp
from jax.experimental import pallas as pl
from jax.experimental.pallas import tpu as pltpu


def _att_bag_kernel(bag_labels_ref, xa_ref, xb_ref, w_ref, b_ref,
                    repre_ref, logits_ref):
    i = pl.program_id(0)
    C = w_ref.shape[0]
    H = w_ref.shape[1]

    def one_bag(lab, x):
        onehot = (jax.lax.broadcasted_iota(jnp.int32, (1, C), 1) == lab
                  ).astype(jnp.float32)
        w = jax.lax.dot_general(
            onehot, w_ref[...], (((1,), (0,)), ((), ())),
            preferred_element_type=jnp.float32,
        )
        logit = jax.lax.dot_general(
            x, w, (((1,), (1,)), ((), ())), preferred_element_type=jnp.float32
        )
        m = jnp.max(logit)
        p = jnp.exp(logit - m)
        s = jnp.sum(p)
        acc = jax.lax.dot_general(
            p, x, (((0,), (0,)), ((), ())), preferred_element_type=jnp.float32
        )
        repre = acc * (1.0 / s)
        row = jax.lax.dot_general(
            repre, w_ref[...], (((1,), (1,)), ((), ())),
            preferred_element_type=jnp.float32,
        ) + b_ref[...]
        return repre, row

    ra, rowa = one_bag(bag_labels_ref[2 * i], xa_ref[...])
    rb, rowb = one_bag(bag_labels_ref[2 * i + 1], xb_ref[...])
    repre_ref[...] = jnp.concatenate([ra, rb], axis=0).reshape(2, 1, H)
    logits_ref[...] = jnp.concatenate([rowa, rowb], axis=0).reshape(2, 1, C)


def kernel(x, labels, scopes, W, b):
    N, H = x.shape
    C = W.shape[0]
    B = scopes.shape[0]
    L = N // B

    starts = jnp.asarray(scopes)[:, 0].astype(jnp.int32)
    bag_labels = jnp.take(labels, starts, axis=0).astype(jnp.int32)
    b2 = b.reshape(1, C)

    grid_spec = pltpu.PrefetchScalarGridSpec(
        num_scalar_prefetch=1,
        grid=(B // 2,),
        in_specs=[
            pl.BlockSpec((L, H), lambda i, *_: (2 * i, 0)),
            pl.BlockSpec((L, H), lambda i, *_: (2 * i + 1, 0)),
            pl.BlockSpec((C, H), lambda i, *_: (0, 0)),
            pl.BlockSpec((1, C), lambda i, *_: (0, 0)),
        ],
        out_specs=[
            pl.BlockSpec((2, 1, H), lambda i, *_: (i, 0, 0)),
            pl.BlockSpec((2, 1, C), lambda i, *_: (i, 0, 0)),
        ],
    )
    repre3, logits3 = pl.pallas_call(
        _att_bag_kernel,
        grid_spec=grid_spec,
        out_shape=[
            jax.ShapeDtypeStruct((B, 1, H), jnp.float32),
            jax.ShapeDtypeStruct((B, 1, C), jnp.float32),
        ],
        compiler_params=pltpu.CompilerParams(
            dimension_semantics=("parallel",)
        ),
    )(bag_labels, x, x, W, b2)
    return (repre3.reshape(B, H), logits3.reshape(B, C))
